# baseline (speedup 1.0000x reference)
import functools

import jax
import jax.numpy as jnp
from jax import lax
from jax.experimental import pallas as pl
from jax.experimental.pallas import tpu as pltpu
from jax.experimental.pallas import tpu_sc as plsc

P = 0.1
SEED = 42

B, L, V = 4096, 50, 128
NC, NS = 2, 16
W = NC * NS
PER_W = B // W
NB = 8
NCHUNK = PER_W // NB


def _binary_entropy(p):
    p = jnp.asarray(p, dtype=jnp.float32)
    q = 1.0 - p
    min_real = jnp.finfo(jnp.float32).min
    log2_p = jnp.maximum(jnp.log2(p), min_real)
    log2_q = jnp.maximum(jnp.log2(q), min_real)
    return -p * log2_p - q * log2_q


def _sc_erase(msg_hbm, mask_hbm, out_hbm, col_hbm,
              buf0, buf1, col0, col1, mask_v,
              isem0, isem1, osem0, osem1, csem0, csem1):
    wid = lax.axis_index("s") * NC + lax.axis_index("c")
    base = wid * PER_W
    iota16 = lax.iota(jnp.int32, 16)
    zeros16 = jnp.zeros((16,), jnp.float32)
    first16b = iota16 == 0
    bufs = (buf0, buf1)
    cols = (col0, col1)
    isems = (isem0, isem1)
    osems = (osem0, osem1)
    csems = (csem0, csem1)

    def in_copy(ci, s):
        b0 = base + ci * NB
        return pltpu.make_async_copy(
            msg_hbm.at[pl.ds(b0, NB)], bufs[s], isems[s])

    def out_copy(ci, s):
        b0 = base + ci * NB
        return pltpu.make_async_copy(
            bufs[s], out_hbm.at[pl.ds(b0, NB), :, 0:V], osems[s])

    def col_copy(ci, s):
        b0 = base + ci * NB
        return pltpu.make_async_copy(
            cols[s], col_hbm.at[pl.ds(lax.div(b0, NB) * 32, 32)], csems[s])

    def process(ci, buf, col):
        b0 = base + ci * NB
        g0 = lax.div(b0, NB) * 32
        pltpu.sync_copy(mask_hbm.at[pl.ds(g0, 32)], mask_v)

        def grp_body(g, c2):
            mv = mask_v[g, pl.ds(0, 16)]
            acc = zeros16
            for j in range(16):
                m = mv[j]
                row = g * 16 + j
                bb = lax.div(row, L)
                ll = lax.rem(row, L)

                p0s = buf[bb, ll, pl.ds(0, 16)][0]

                @pl.when(m != 0.0)
                def _():
                    for k in range(8):
                        buf[bb, ll, pl.ds(k * 16, 16)] = zeros16

                t2 = buf[bb, ll, pl.ds(0, 16)]
                p0f = lax.full((16,), p0s, jnp.float32)
                buf[bb, ll, pl.ds(0, 16)] = jnp.where(first16b, p0f, t2)

                lvf = lax.full((16,), m * (1.0 - p0s), jnp.float32)
                acc = jnp.where(iota16 == j, lvf, acc)
            col[g, pl.ds(0, 16)] = acc
            return c2

        lax.fori_loop(0, NB * L // 16, grp_body, 0)

    in_copy(0, 0).start()
    for ci in range(NCHUNK):
        s = ci % 2
        if ci + 1 < NCHUNK:
            s2 = (ci + 1) % 2
            if ci >= 1:
                out_copy(ci - 1, s2).wait()
                col_copy(ci - 1, s2).wait()
            in_copy(ci + 1, s2).start()
        in_copy(ci, s).wait()
        process(ci, bufs[s], cols[s])
        out_copy(ci, s).start()
        col_copy(ci, s).start()
    out_copy(NCHUNK - 2, (NCHUNK - 2) % 2).wait()
    col_copy(NCHUNK - 2, (NCHUNK - 2) % 2).wait()
    out_copy(NCHUNK - 1, (NCHUNK - 1) % 2).wait()
    col_copy(NCHUNK - 1, (NCHUNK - 1) % 2).wait()


_sc_call = functools.partial(
    pl.kernel,
    mesh=plsc.VectorSubcoreMesh(core_axis_name="c", subcore_axis_name="s"),
    out_type=(
        jax.ShapeDtypeStruct((B, L, V + 1), jnp.float32),
        jax.ShapeDtypeStruct((B // NB * 32, 16), jnp.float32),
    ),
    scratch_types=[
        pltpu.VMEM((NB, L, V), jnp.float32),
        pltpu.VMEM((NB, L, V), jnp.float32),
        pltpu.VMEM((32, 16), jnp.float32),
        pltpu.VMEM((32, 16), jnp.float32),
        pltpu.VMEM((32, 16), jnp.float32),
        pltpu.SemaphoreType.DMA,
        pltpu.SemaphoreType.DMA,
        pltpu.SemaphoreType.DMA,
        pltpu.SemaphoreType.DMA,
        pltpu.SemaphoreType.DMA,
        pltpu.SemaphoreType.DMA,
    ],
)(_sc_erase)


def _ent_kernel(h_ref, ent_ref, ent_out_ref):
    ent_out_ref[...] = ent_ref[...] + h_ref[0]


@jax.jit
def _run(messages, entropy, apply_noise):
    noise_on = (jnp.asarray(apply_noise) != 0)
    target_mask = jax.random.uniform(jax.random.key(SEED), (B, L)) < P
    mask_f = (target_mask & noise_on).astype(jnp.float32)
    h = jnp.where(noise_on, _binary_entropy(P), 0.0).reshape(1)

    mask2 = mask_f.reshape(B // NB, NB * L // 16, 16)
    mask2 = jnp.pad(mask2, ((0, 0), (0, 32 - NB * L // 16), (0, 0)))
    mask2 = mask2.reshape(B // NB * 32, 16)
    probs_main, col_out = _sc_call(messages, mask2)
    lastcol = col_out.reshape(B // NB, 32, 16)[:, :NB * L // 16, :]
    probs_out = probs_main.at[:, :, V].set(lastcol.reshape(B, L))

    bm = 512
    ent_out = pl.pallas_call(
        _ent_kernel,
        grid_spec=pltpu.PrefetchScalarGridSpec(
            num_scalar_prefetch=1,
            grid=(B // bm,),
            in_specs=[pl.BlockSpec((bm, L), lambda i, h: (i, 0))],
            out_specs=pl.BlockSpec((bm, L), lambda i, h: (i, 0)),
        ),
        out_shape=jax.ShapeDtypeStruct((B, L), entropy.dtype),
    )(h, entropy)
    return probs_out, ent_out


def kernel(messages, entropy, apply_noise):
    return _run(messages, entropy, apply_noise)
